# baseline (speedup 1.0000x reference)
import jax
import jax.numpy as jnp
from jax import lax
from jax.experimental import pallas as pl
from jax.experimental.pallas import tpu as pltpu
from jax.experimental.pallas import tpu_sc as plsc

_C = 1000
_P = 16
_D = 512
_ROWS = _C * _P
_NC = 2
_NS = 16
_NW = _NC * _NS
_RPW = _ROWS // _NW


def _body(protos_hbm, out_hbm, idx_hbm, idx_v):
    wid = lax.axis_index("s") * _NC + lax.axis_index("c")
    base = 31 * wid + jnp.maximum(wid - 24, 0)
    pltpu.sync_copy(protos_hbm.at[pl.ds(base, 31)],
                    out_hbm.at[pl.ds(base, 31)])

    @pl.when(wid >= 24)
    def _():
        pltpu.sync_copy(protos_hbm.at[pl.ds(base + 31, 1)],
                        out_hbm.at[pl.ds(base + 31, 1)])

    @pl.when(wid == 0)
    def _():
        def fill(c, carry):
            idx_v[pl.ds(c * _P, _P)] = jnp.full((_P,), c, jnp.int32)
            return carry
        lax.fori_loop(0, _C, fill, 0)
        pltpu.sync_copy(idx_v, idx_hbm)


def kernel(vision_protos):
    f = pl.kernel(
        _body,
        out_type=(jax.ShapeDtypeStruct((_C, _P, _D), jnp.float32),
                  jax.ShapeDtypeStruct((_ROWS,), jnp.int32)),
        mesh=plsc.VectorSubcoreMesh(core_axis_name="c", subcore_axis_name="s"),
        scratch_types=[pltpu.VMEM((_ROWS,), jnp.int32)],
    )
    stacked, class_idx = f(vision_protos)
    return (stacked.reshape(_ROWS, _D), class_idx)


# speedup vs baseline: 21.9152x; 21.9152x over previous
import jax
import jax.numpy as jnp
from jax import lax
from jax.experimental import pallas as pl
from jax.experimental.pallas import tpu as pltpu
from jax.experimental.pallas import tpu_sc as plsc

_C = 1000
_P = 16
_D = 512
_ROWS = _C * _P
_NC = 2
_NS = 16
_NW = _NC * _NS
_CHUNK = 4
_NCHUNKS = _C // _CHUNK
_JMAX = -(-_NCHUNKS // _NW)


def _body(protos_hbm, out_hbm, idx_hbm, buf, idx_v, r0, r1, w0, w1):
    wid = lax.axis_index("s") * _NC + lax.axis_index("c")
    rsems = (r0, r1)
    wsems = (w0, w1)

    def rd(j, b):
        c0 = _CHUNK * (wid + _NW * j)
        return pltpu.make_async_copy(protos_hbm.at[pl.ds(c0, _CHUNK)],
                                     buf.at[b], rsems[b])

    def wr(j, b):
        c0 = _CHUNK * (wid + _NW * j)
        return pltpu.make_async_copy(buf.at[b],
                                     out_hbm.at[pl.ds(c0, _CHUNK)], wsems[b])

    last_ok = wid < (_NCHUNKS - _NW * (_JMAX - 1))

    def guarded(j, mk):
        if j == _JMAX - 1:
            @pl.when(last_ok)
            def _():
                mk()
        else:
            mk()

    guarded(0, lambda: rd(0, 0).start())

    for i in range(32):
        idx_v[pl.ds(_P * i, _P)] = jnp.full((_P,), 32 * wid + i, jnp.int32)

    @pl.when(wid < _NW - 1)
    def _():
        pltpu.sync_copy(idx_v, idx_hbm.at[pl.ds(512 * wid, 512)])

    @pl.when(wid == _NW - 1)
    def _():
        pltpu.sync_copy(idx_v.at[pl.ds(0, 128)],
                        idx_hbm.at[pl.ds(512 * (_NW - 1), 128)])

    for j in range(_JMAX):
        b = j % 2
        guarded(j, lambda: rd(j, b).wait())
        if j >= 1:
            wr(j - 1, 1 - b).wait()
        if j + 1 < _JMAX:
            guarded(j + 1, lambda: rd(j + 1, 1 - b).start())
        guarded(j, lambda: wr(j, b).start())
    guarded(_JMAX - 1, lambda: wr(_JMAX - 1, (_JMAX - 1) % 2).wait())


def kernel(vision_protos):
    f = pl.kernel(
        _body,
        out_type=(jax.ShapeDtypeStruct((_C, _P, _D), jnp.float32),
                  jax.ShapeDtypeStruct((_ROWS,), jnp.int32)),
        mesh=plsc.VectorSubcoreMesh(core_axis_name="c", subcore_axis_name="s"),
        scratch_types=[
            pltpu.VMEM((2, _CHUNK, _P, _D), jnp.float32),
            pltpu.VMEM((512,), jnp.int32),
            pltpu.SemaphoreType.DMA,
            pltpu.SemaphoreType.DMA,
            pltpu.SemaphoreType.DMA,
            pltpu.SemaphoreType.DMA,
        ],
    )
    stacked, class_idx = f(vision_protos)
    return (stacked.reshape(_ROWS, _D), class_idx)
